# Initial kernel scaffold; baseline (speedup 1.0000x reference)
import jax
import jax.numpy as jnp
from jax.experimental import pallas as pl


def kernel(x, edge_index, edge_weight, W_lin, edge_table, W_heads, a_src, a_dst):
    raise NotImplementedError("write your pallas kernel here")


# speedup vs baseline: 15.4222x; 15.4222x over previous
import functools

import jax
import jax.numpy as jnp
from jax import lax
from jax.experimental import pallas as pl
from jax.experimental.pallas import tpu as pltpu
from jax.experimental.pallas import tpu_sc as plsc

N = 10000
NP = 10240
E = 320000
D = 128
HD = D // 2
NH = 3
NEG = 0.2
EV = 22754

NC = 2
NS = 16
NW = NC * NS
EPW = E // NW
K = 80
NG = K // 16
NCHUNK = EPW // K
RPT = NP // NS
DW = NP * 4
DWPT = DW // NS
NPAIR = NG * NH
PPS = 5


def _dense_body(x_ref, wlin_ref, wh_ref, asrc_ref, adst_ref,
                hlo_ref, hhi_ref, alpha_ref):
    xb = jnp.dot(x_ref[...], wlin_ref[...], preferred_element_type=jnp.float32)
    feats = []
    for h in range(NH):
        feats.append(jnp.dot(xb, wh_ref[h], preferred_element_type=jnp.float32))
    hlo_ref[...] = jnp.concatenate([f[:, 0:HD] for f in feats], axis=1)
    hhi_ref[...] = jnp.concatenate([f[:, HD:D] for f in feats], axis=1)
    cols = []
    for h in range(NH):
        cols.append(jnp.sum(feats[h] * asrc_ref[h][None, :], axis=1, keepdims=True))
    for h in range(NH):
        cols.append(jnp.sum(feats[h] * adst_ref[h][None, :], axis=1, keepdims=True))
    cols.append(jnp.zeros_like(cols[0]))
    cols.append(jnp.zeros_like(cols[0]))
    alpha_ref[...] = jnp.concatenate(cols, axis=1)


def _dense(x, W_lin, W_heads, a_src, a_dst):
    BN = 1000
    return pl.pallas_call(
        _dense_body,
        grid=(N // BN,),
        in_specs=[
            pl.BlockSpec((BN, D), lambda i: (i, 0)),
            pl.BlockSpec((D, D), lambda i: (0, 0)),
            pl.BlockSpec((NH, D, D), lambda i: (0, 0, 0)),
            pl.BlockSpec((NH, D), lambda i: (0, 0)),
            pl.BlockSpec((NH, D), lambda i: (0, 0)),
        ],
        out_specs=[
            pl.BlockSpec((BN, NH * HD), lambda i: (i, 0)),
            pl.BlockSpec((BN, NH * HD), lambda i: (i, 0)),
            pl.BlockSpec((BN, 8), lambda i: (i, 0)),
        ],
        out_shape=[
            jax.ShapeDtypeStruct((N, NH * HD), jnp.float32),
            jax.ShapeDtypeStruct((N, NH * HD), jnp.float32),
            jax.ShapeDtypeStruct((N, 8), jnp.float32),
        ],
    )(x, W_lin, W_heads, a_src, a_dst)


def _etsplit_body(et_ref, lo_ref, hi_ref):
    v = et_ref[...]
    lo_ref[...] = v[:, 0:HD]
    hi_ref[...] = v[:, HD:D]


def _etsplit(edge_table):
    BV = 1024
    return pl.pallas_call(
        _etsplit_body,
        grid=(pl.cdiv(EV, BV),),
        in_specs=[pl.BlockSpec((BV, D), lambda i: (i, 0))],
        out_specs=[
            pl.BlockSpec((BV, HD), lambda i: (i, 0)),
            pl.BlockSpec((BV, HD), lambda i: (i, 0)),
        ],
        out_shape=[
            jax.ShapeDtypeStruct((EV, HD), jnp.float32),
            jax.ShapeDtypeStruct((EV, HD), jnp.float32),
        ],
    )(edge_table)


def _mesh():
    return plsc.VectorSubcoreMesh(core_axis_name="c", subcore_axis_name="s")


def _pass1_body(src_hbm, dst_hbm, alpha_hbm,
                e0_hbm, e1_hbm, e2_hbm, denp_hbm,
                atbl, srcb, dstb, e0b, e1b, e2b,
                ev0, ev1, ev2, ix0, ix1, ix2, zb, dsh):
    cid = lax.axis_index("c")
    sid = lax.axis_index("s")
    wid = sid * NC + cid
    base = wid * EPW
    ebufs = (e0b, e1b, e2b)
    ehbms = (e0_hbm, e1_hbm, e2_hbm)
    evbs = (ev0, ev1, ev2)
    ixbs = (ix0, ix1, ix2)

    zero16 = jnp.zeros((16,), jnp.float32)

    def zrow(i, _):
        zb[pl.ds(i * 16, 16)] = zero16
        return 0
    lax.fori_loop(0, DWPT // 2 // 16, zrow, 0)

    pltpu.sync_copy(zb, dsh.at[pl.ds(sid * DWPT, DWPT // 2)])
    pltpu.sync_copy(zb, dsh.at[pl.ds(sid * DWPT + DWPT // 2, DWPT // 2)])
    plsc.subcore_barrier()

    pltpu.sync_copy(alpha_hbm, atbl)

    def chunk(c, _):
        b = base + c * K
        pltpu.sync_copy(src_hbm.at[pl.ds(b, K)], srcb)
        pltpu.sync_copy(dst_hbm.at[pl.ds(b, K)], dstb)
        for g in range(NG):
            sv8 = srcb[pl.ds(g * 16, 16)] * 8
            dv = dstb[pl.ds(g * 16, 16)]
            dv8 = dv * 8
            dv4 = dv * 4
            for h in range(NH):
                a_s = plsc.load_gather(atbl, [sv8 + h])
                a_d = plsc.load_gather(atbl, [dv8 + (3 + h)])
                z = a_s + a_d
                ev = jnp.exp(jnp.maximum(z, NEG * z))
                ebufs[h][pl.ds(c * K + g * 16, 16)] = ev
                p = g * NH + h
                bp, slot = p // PPS, p % PPS
                evbs[bp][pl.ds(slot * 16, 16)] = ev
                ixbs[bp][pl.ds(slot * 16, 16)] = dv4 + h
        for i in range(NPAIR // PPS):
            pltpu.sync_copy(evbs[i], dsh.at[ixbs[i]], add=True)
        return 0

    lax.fori_loop(0, NCHUNK, chunk, 0)

    for h in range(NH):
        pltpu.sync_copy(ebufs[h], ehbms[h].at[pl.ds(base, EPW)])

    plsc.subcore_barrier()
    pltpu.sync_copy(dsh.at[pl.ds(sid * DWPT, DWPT)],
                    denp_hbm.at[pl.ds(cid * DW + sid * DWPT, DWPT)])


def _pass1(src, dst, alpha_flat):
    f = functools.partial(
        pl.kernel,
        out_type=(
            jax.ShapeDtypeStruct((E,), jnp.float32),
            jax.ShapeDtypeStruct((E,), jnp.float32),
            jax.ShapeDtypeStruct((E,), jnp.float32),
            jax.ShapeDtypeStruct((NC * DW,), jnp.float32),
        ),
        mesh=_mesh(),
        scratch_types=[
            pltpu.VMEM((N * 8,), jnp.float32),
            pltpu.VMEM((K,), jnp.int32),
            pltpu.VMEM((K,), jnp.int32),
            pltpu.VMEM((EPW,), jnp.float32),
            pltpu.VMEM((EPW,), jnp.float32),
            pltpu.VMEM((EPW,), jnp.float32),
            pltpu.VMEM((PPS * 16,), jnp.float32),
            pltpu.VMEM((PPS * 16,), jnp.float32),
            pltpu.VMEM((PPS * 16,), jnp.float32),
            pltpu.VMEM((PPS * 16,), jnp.int32),
            pltpu.VMEM((PPS * 16,), jnp.int32),
            pltpu.VMEM((PPS * 16,), jnp.int32),
            pltpu.VMEM((DWPT // 2,), jnp.float32),
            pltpu.VMEM_SHARED((DW,), jnp.float32),
        ],
        compiler_params=pltpu.CompilerParams(needs_layout_passes=False),
    )(_pass1_body)
    return f(src, dst, alpha_flat)


def _rden_body(denp_ref, rden_ref):
    s = denp_ref[0:DW] + denp_ref[DW:2 * DW]
    rden_ref[...] = 1.0 / (s + 1e-16)


def _rden(denp):
    return pl.pallas_call(
        _rden_body,
        out_shape=jax.ShapeDtypeStruct((DW,), jnp.float32),
    )(denp)


def _pass2_body(src_hbm, dst_hbm, ewi_hbm, e0_hbm, e1_hbm, e2_hbm,
                h_hbm, et_hbm, rden_hbm, outp_hbm,
                rtbl, srcb, dstb, ewib, e0b, e1b, e2b, w0b, w1b, w2b,
                hbuf, ewbuf, msgbuf, osh, sem1, sem2):
    cid = lax.axis_index("c")
    sid = lax.axis_index("s")
    wid = sid * NC + cid
    base = wid * EPW
    ebufs = (e0b, e1b, e2b)
    ehbms = (e0_hbm, e1_hbm, e2_hbm)
    wbufs = (w0b, w1b, w2b)

    zero16 = jnp.zeros((16,), jnp.float32)

    def zrow(i, _):
        for q in range(HD // 16):
            msgbuf[i, pl.ds(q * 16, 16)] = zero16
        return 0
    lax.fori_loop(0, K, zrow, 0)

    for j in range(RPT // K):
        pltpu.sync_copy(msgbuf, osh.at[pl.ds(sid * RPT + j * K, K)])
    plsc.subcore_barrier()

    pltpu.sync_copy(rden_hbm, rtbl)

    def chunk(c, _):
        b = base + c * K
        pltpu.sync_copy(src_hbm.at[pl.ds(b, K)], srcb)
        pltpu.sync_copy(dst_hbm.at[pl.ds(b, K)], dstb)
        pltpu.sync_copy(ewi_hbm.at[pl.ds(b, K)], ewib)
        for h in range(NH):
            pltpu.sync_copy(ehbms[h].at[pl.ds(b, K)], ebufs[h])
        cp1 = pltpu.async_copy(h_hbm.at[srcb], hbuf, sem1)
        cp2 = pltpu.async_copy(et_hbm.at[ewib], ewbuf, sem2)
        for g in range(NG):
            dv4 = dstb[pl.ds(g * 16, 16)] * 4
            for h in range(NH):
                rd = plsc.load_gather(rtbl, [dv4 + h])
                wbufs[h][pl.ds(g * 16, 16)] = \
                    ebufs[h][pl.ds(g * 16, 16)] * rd * jnp.float32(1.0 / NH)
        cp1.wait()
        cp2.wait()

        for g in range(NG):
            wv0 = w0b[pl.ds(g * 16, 16)]
            wv1 = w1b[pl.ds(g * 16, 16)]
            wv2 = w2b[pl.ds(g * 16, 16)]
            for kk in range(16):
                k = g * 16 + kk
                w0 = wv0[kk]
                w1 = wv1[kk]
                w2 = wv2[kk]
                for q in range(HD // 16):
                    sl = pl.ds(q * 16, 16)
                    m = (hbuf[k, pl.ds(q * 16, 16)] * w0
                         + hbuf[k, pl.ds(HD + q * 16, 16)] * w1
                         + hbuf[k, pl.ds(2 * HD + q * 16, 16)] * w2)
                    msgbuf[k, sl] = m * ewbuf[k, sl]
        pltpu.sync_copy(msgbuf, osh.at[dstb], add=True)
        return 0

    lax.fori_loop(0, NCHUNK, chunk, 0)

    plsc.subcore_barrier()
    pltpu.sync_copy(osh.at[pl.ds(sid * RPT, RPT)],
                    outp_hbm.at[pl.ds(cid * NP + sid * RPT, RPT)])


def _pass2(src, dst, ewi, e0, e1, e2, Hhalf, et_half, rden):
    f = functools.partial(
        pl.kernel,
        out_type=jax.ShapeDtypeStruct((NC * NP, HD), jnp.float32),
        mesh=_mesh(),
        scratch_types=[
            pltpu.VMEM((DW,), jnp.float32),
            pltpu.VMEM((K,), jnp.int32),
            pltpu.VMEM((K,), jnp.int32),
            pltpu.VMEM((K,), jnp.int32),
            pltpu.VMEM((K,), jnp.float32),
            pltpu.VMEM((K,), jnp.float32),
            pltpu.VMEM((K,), jnp.float32),
            pltpu.VMEM((K,), jnp.float32),
            pltpu.VMEM((K,), jnp.float32),
            pltpu.VMEM((K,), jnp.float32),
            pltpu.VMEM((K, NH * HD), jnp.float32),
            pltpu.VMEM((K, HD), jnp.float32),
            pltpu.VMEM((K, HD), jnp.float32),
            pltpu.VMEM_SHARED((NP, HD), jnp.float32),
            pltpu.SemaphoreType.DMA,
            pltpu.SemaphoreType.DMA,
        ],
        compiler_params=pltpu.CompilerParams(
            needs_layout_passes=False, use_tc_tiling_on_sc=False),
    )(_pass2_body)
    return f(src, dst, ewi, e0, e1, e2, Hhalf, et_half, rden)


def _final_body(lo_ref, hi_ref, out_ref):
    lo = lo_ref[0:N, :] + lo_ref[NP:NP + N, :]
    hi = hi_ref[0:N, :] + hi_ref[NP:NP + N, :]
    out_ref[...] = jnp.concatenate([lo, hi], axis=1)


def _final(outp_lo, outp_hi):
    return pl.pallas_call(
        _final_body,
        out_shape=jax.ShapeDtypeStruct((N, D), jnp.float32),
    )(outp_lo, outp_hi)


def kernel(x, edge_index, edge_weight, W_lin, edge_table, W_heads, a_src, a_dst):
    ei = edge_index.astype(jnp.int32)
    src = ei[0]
    dst = ei[1]
    ewi = edge_weight.astype(jnp.int32)
    Hlo, Hhi, alpha = _dense(x, W_lin, W_heads, a_src, a_dst)
    etlo, ethi = _etsplit(edge_table)
    e0, e1, e2, denp = _pass1(src, dst, alpha.reshape(N * 8))
    rden = _rden(denp)
    outp_lo = _pass2(src, dst, ewi, e0, e1, e2, Hlo, etlo, rden)
    outp_hi = _pass2(src, dst, ewi, e0, e1, e2, Hhi, ethi, rden)
    return _final(outp_lo, outp_hi)


# speedup vs baseline: 31.2460x; 2.0260x over previous
import functools

import jax
import jax.numpy as jnp
from jax import lax
from jax.experimental import pallas as pl
from jax.experimental.pallas import tpu as pltpu
from jax.experimental.pallas import tpu_sc as plsc

N = 10000
NP = 10240
E = 320000
D = 128
HD = D // 2
NH = 3
NEG = 0.2
EV = 22754

NC = 2
NS = 16
NW = NC * NS
EPW = E // NW
K = 80
NG = K // 16
NCHUNK = EPW // K
RPT = NP // NS
DW = NP * 4
DWPT = DW // NS
NPAIR = NG * NH
PPS = 5


def _dense_body(x_ref, wlin_ref, wh_ref, asrc_ref, adst_ref,
                hlo_ref, hhi_ref, alpha_ref):
    xb = jnp.dot(x_ref[...], wlin_ref[...], preferred_element_type=jnp.float32)
    feats = []
    for h in range(NH):
        feats.append(jnp.dot(xb, wh_ref[h], preferred_element_type=jnp.float32))
    hlo_ref[...] = jnp.concatenate([f[:, 0:HD] for f in feats], axis=1)
    hhi_ref[...] = jnp.concatenate([f[:, HD:D] for f in feats], axis=1)
    cols = []
    for h in range(NH):
        cols.append(jnp.sum(feats[h] * asrc_ref[h][None, :], axis=1, keepdims=True))
    for h in range(NH):
        cols.append(jnp.sum(feats[h] * adst_ref[h][None, :], axis=1, keepdims=True))
    alpha_ref[...] = jnp.concatenate(cols, axis=1)


def _dense(x, W_lin, W_heads, a_src, a_dst):
    BN = 1000
    return pl.pallas_call(
        _dense_body,
        grid=(N // BN,),
        in_specs=[
            pl.BlockSpec((BN, D), lambda i: (i, 0)),
            pl.BlockSpec((D, D), lambda i: (0, 0)),
            pl.BlockSpec((NH, D, D), lambda i: (0, 0, 0)),
            pl.BlockSpec((NH, D), lambda i: (0, 0)),
            pl.BlockSpec((NH, D), lambda i: (0, 0)),
        ],
        out_specs=[
            pl.BlockSpec((BN, NH * HD), lambda i: (i, 0)),
            pl.BlockSpec((BN, NH * HD), lambda i: (i, 0)),
            pl.BlockSpec((BN, 6), lambda i: (i, 0)),
        ],
        out_shape=[
            jax.ShapeDtypeStruct((N, NH * HD), jnp.float32),
            jax.ShapeDtypeStruct((N, NH * HD), jnp.float32),
            jax.ShapeDtypeStruct((N, 6), jnp.float32),
        ],
    )(x, W_lin, W_heads, a_src, a_dst)


def _etsplit_body(et_ref, lo_ref, hi_ref):
    v = et_ref[...]
    lo_ref[...] = v[:, 0:HD]
    hi_ref[...] = v[:, HD:D]


def _etsplit(edge_table):
    BV = 1024
    return pl.pallas_call(
        _etsplit_body,
        grid=(pl.cdiv(EV, BV),),
        in_specs=[pl.BlockSpec((BV, D), lambda i: (i, 0))],
        out_specs=[
            pl.BlockSpec((BV, HD), lambda i: (i, 0)),
            pl.BlockSpec((BV, HD), lambda i: (i, 0)),
        ],
        out_shape=[
            jax.ShapeDtypeStruct((EV, HD), jnp.float32),
            jax.ShapeDtypeStruct((EV, HD), jnp.float32),
        ],
    )(edge_table)


def _mesh():
    return plsc.VectorSubcoreMesh(core_axis_name="c", subcore_axis_name="s")


def _pass1_body(src_hbm, dst_hbm, alpha_hbm,
                e0_hbm, e1_hbm, e2_hbm, denp_hbm,
                atbl, srcbs, dstbs, e0b, e1b, e2b,
                evbss, ixbss, zb, dsh, isems, ssems):
    cid = lax.axis_index("c")
    sid = lax.axis_index("s")
    wid = sid * NC + cid
    base = wid * EPW
    ebufs = (e0b, e1b, e2b)
    ehbms = (e0_hbm, e1_hbm, e2_hbm)

    zero16 = jnp.zeros((16,), jnp.float32)

    def zrow(i, _):
        zb[pl.ds(i * 16, 16)] = zero16
        return 0
    lax.fori_loop(0, DWPT // 2 // 16, zrow, 0)

    pltpu.sync_copy(zb, dsh.at[pl.ds(sid * DWPT, DWPT // 2)])
    pltpu.sync_copy(zb, dsh.at[pl.ds(sid * DWPT + DWPT // 2, DWPT // 2)])
    plsc.subcore_barrier()

    pltpu.sync_copy(alpha_hbm, atbl)

    def issue_idx(c, p):
        b = base + c * K
        pltpu.async_copy(src_hbm.at[pl.ds(b, K)], srcbs[p], isems[p])
        pltpu.async_copy(dst_hbm.at[pl.ds(b, K)], dstbs[p], isems[p])

    def wait_idx(p):
        pltpu.make_async_copy(src_hbm.at[pl.ds(0, K)], srcbs[p], isems[p]).wait()
        pltpu.make_async_copy(dst_hbm.at[pl.ds(0, K)], dstbs[p], isems[p]).wait()

    def wait_scat(p):
        for h in range(NH):
            pltpu.make_async_copy(
                evbss[p][h], dsh.at[ixbss[p][h]], ssems[p]).wait()

    def step(c, p, first, last):
        wait_idx(p)
        if not first:
            wait_scat(p)

        def comp(g, _):
            sv6 = srcbs[p][pl.ds(g * 16, 16)] * 6
            dv = dstbs[p][pl.ds(g * 16, 16)]
            dv6 = dv * 6
            dv4 = dv * 4
            for h in range(NH):
                a_s = plsc.load_gather(atbl, [sv6 + h])
                a_d = plsc.load_gather(atbl, [dv6 + (3 + h)])
                z = a_s + a_d
                ev = jnp.exp(jnp.maximum(z, NEG * z))
                ebufs[h][pl.ds(c * K + g * 16, 16)] = ev
                evbss[p][h][pl.ds(g * 16, 16)] = ev
                ixbss[p][h][pl.ds(g * 16, 16)] = dv4 + h
            return 0
        lax.fori_loop(0, NG, comp, 0)
        for h in range(NH):
            pltpu.async_copy(evbss[p][h], dsh.at[ixbss[p][h]], ssems[p],
                             add=True)
        if not last:
            @pl.when(c + 2 < NCHUNK)
            def _():
                issue_idx(c + 2, p)

    issue_idx(0, 0)
    issue_idx(1, 1)
    step(0, 0, True, False)
    step(1, 1, True, False)

    def loop(i, _):
        c = 2 * i
        step(c, 0, False, False)
        step(c + 1, 1, False, False)
        return 0

    lax.fori_loop(1, (NCHUNK - 1) // 2, loop, 0)
    step(NCHUNK - 1, 0, False, True)
    wait_scat(1)
    wait_scat(0)

    for h in range(NH):
        pltpu.sync_copy(ebufs[h], ehbms[h].at[pl.ds(base, EPW)])

    plsc.subcore_barrier()
    pltpu.sync_copy(dsh.at[pl.ds(sid * DWPT, DWPT)],
                    denp_hbm.at[pl.ds(cid * DW + sid * DWPT, DWPT)])


def _pass1(src, dst, alpha_flat):
    f = functools.partial(
        pl.kernel,
        out_type=(
            jax.ShapeDtypeStruct((E,), jnp.float32),
            jax.ShapeDtypeStruct((E,), jnp.float32),
            jax.ShapeDtypeStruct((E,), jnp.float32),
            jax.ShapeDtypeStruct((NC * DW,), jnp.float32),
        ),
        mesh=_mesh(),
        scratch_types=[
            pltpu.VMEM((N * 6,), jnp.float32),
            [pltpu.VMEM((K,), jnp.int32) for _ in range(2)],
            [pltpu.VMEM((K,), jnp.int32) for _ in range(2)],
            pltpu.VMEM((EPW,), jnp.float32),
            pltpu.VMEM((EPW,), jnp.float32),
            pltpu.VMEM((EPW,), jnp.float32),
            [[pltpu.VMEM((PPS * 16,), jnp.float32) for _ in range(3)]
             for _ in range(2)],
            [[pltpu.VMEM((PPS * 16,), jnp.int32) for _ in range(3)]
             for _ in range(2)],
            pltpu.VMEM((DWPT // 2,), jnp.float32),
            pltpu.VMEM_SHARED((DW,), jnp.float32),
            [pltpu.SemaphoreType.DMA for _ in range(2)],
            [pltpu.SemaphoreType.DMA for _ in range(2)],
        ],
        compiler_params=pltpu.CompilerParams(needs_layout_passes=False),
    )(_pass1_body)
    return f(src, dst, alpha_flat)


def _rden_body(denp_ref, rden_ref):
    s = denp_ref[0:DW] + denp_ref[DW:2 * DW]
    rden_ref[...] = 1.0 / (s + 1e-16)


def _rden(denp):
    return pl.pallas_call(
        _rden_body,
        out_shape=jax.ShapeDtypeStruct((DW,), jnp.float32),
    )(denp)


def _pass2_body(src_hbm, dst_hbm, ewi_hbm, e0_hbm, e1_hbm, e2_hbm,
                h_hbm, et_hbm, rden_hbm, outp_hbm,
                rtbl, srcbs, dstbs, ewibs, ebss, sdst, w0b, w1b, w2b,
                hbufs, ewbufs, msgbuf, osh, isems, gsems, ssem):
    cid = lax.axis_index("c")
    sid = lax.axis_index("s")
    wid = sid * NC + cid
    base = wid * EPW
    wbufs = (w0b, w1b, w2b)
    ehbms = (e0_hbm, e1_hbm, e2_hbm)

    zero16 = jnp.zeros((16,), jnp.float32)

    def zrow(i, _):
        for q in range(HD // 16):
            msgbuf[i, pl.ds(q * 16, 16)] = zero16
        return 0
    lax.fori_loop(0, K, zrow, 0)

    for j in range(RPT // K):
        pltpu.sync_copy(msgbuf, osh.at[pl.ds(sid * RPT + j * K, K)])
    plsc.subcore_barrier()

    pltpu.sync_copy(rden_hbm, rtbl)

    def issue_idx(c, p):
        b = base + c * K
        pltpu.async_copy(src_hbm.at[pl.ds(b, K)], srcbs[p], isems[p])
        pltpu.async_copy(dst_hbm.at[pl.ds(b, K)], dstbs[p], isems[p])
        pltpu.async_copy(ewi_hbm.at[pl.ds(b, K)], ewibs[p], isems[p])
        for h in range(NH):
            pltpu.async_copy(ehbms[h].at[pl.ds(b, K)], ebss[p][h], isems[p])

    def wait_idx(p):
        pltpu.make_async_copy(src_hbm.at[pl.ds(0, K)], srcbs[p], isems[p]).wait()
        pltpu.make_async_copy(dst_hbm.at[pl.ds(0, K)], dstbs[p], isems[p]).wait()
        pltpu.make_async_copy(ewi_hbm.at[pl.ds(0, K)], ewibs[p], isems[p]).wait()
        for h in range(NH):
            pltpu.make_async_copy(
                ehbms[h].at[pl.ds(0, K)], ebss[p][h], isems[p]).wait()

    def issue_gath(p):
        pltpu.async_copy(h_hbm.at[srcbs[p]], hbufs[p], gsems[p])
        pltpu.async_copy(et_hbm.at[ewibs[p]], ewbufs[p], gsems[p])

    def wait_gath(p):
        pltpu.make_async_copy(h_hbm.at[srcbs[p]], hbufs[p], gsems[p]).wait()
        pltpu.make_async_copy(et_hbm.at[ewibs[p]], ewbufs[p], gsems[p]).wait()

    def wait_scat():
        pltpu.make_async_copy(msgbuf, osh.at[sdst], ssem).wait()

    def step(c, p, first, last):
        wait_gath(p)

        def wcomp(g, _):
            dv = dstbs[p][pl.ds(g * 16, 16)]
            dv4 = dv * 4
            for h in range(NH):
                rd = plsc.load_gather(rtbl, [dv4 + h])
                wbufs[h][pl.ds(g * 16, 16)] = \
                    ebss[p][h][pl.ds(g * 16, 16)] * rd * jnp.float32(1.0 / NH)
            return 0
        lax.fori_loop(0, NG, wcomp, 0)
        if not first:
            wait_scat()

        def cpd(g, _):
            sdst[pl.ds(g * 16, 16)] = dstbs[p][pl.ds(g * 16, 16)]
            return 0
        lax.fori_loop(0, NG, cpd, 0)
        if not last:
            @pl.when(c + 2 < NCHUNK)
            def _():
                issue_idx(c + 2, p)
            wait_idx(1 - p)
            issue_gath(1 - p)
        hbuf = hbufs[p]
        ewbuf = ewbufs[p]

        def mcomp(g, _):
            wv0 = w0b[pl.ds(g * 16, 16)]
            wv1 = w1b[pl.ds(g * 16, 16)]
            wv2 = w2b[pl.ds(g * 16, 16)]
            for kk in range(16):
                k = g * 16 + kk
                w0 = wv0[kk]
                w1 = wv1[kk]
                w2 = wv2[kk]
                for q in range(HD // 16):
                    sl = pl.ds(q * 16, 16)
                    m = (hbuf[k, pl.ds(q * 16, 16)] * w0
                         + hbuf[k, pl.ds(HD + q * 16, 16)] * w1
                         + hbuf[k, pl.ds(2 * HD + q * 16, 16)] * w2)
                    msgbuf[k, sl] = m * ewbuf[k, sl]
            return 0
        lax.fori_loop(0, NG, mcomp, 0)
        pltpu.async_copy(msgbuf, osh.at[sdst], ssem, add=True)

    issue_idx(0, 0)
    issue_idx(1, 1)
    wait_idx(0)
    issue_gath(0)
    step(0, 0, True, False)
    step(1, 1, False, False)

    def loop(i, _):
        c = 2 * i
        step(c, 0, False, False)
        step(c + 1, 1, False, False)
        return 0

    lax.fori_loop(1, (NCHUNK - 1) // 2, loop, 0)
    step(NCHUNK - 1, 0, False, True)
    wait_scat()

    plsc.subcore_barrier()
    pltpu.sync_copy(osh.at[pl.ds(sid * RPT, RPT)],
                    outp_hbm.at[pl.ds(cid * NP + sid * RPT, RPT)])


def _pass2(src, dst, ewi, e0, e1, e2, Hhalf, et_half, rden):
    f = functools.partial(
        pl.kernel,
        out_type=jax.ShapeDtypeStruct((NC * NP, HD), jnp.float32),
        mesh=_mesh(),
        scratch_types=[
            pltpu.VMEM((DW,), jnp.float32),
            [pltpu.VMEM((K,), jnp.int32) for _ in range(2)],
            [pltpu.VMEM((K,), jnp.int32) for _ in range(2)],
            [pltpu.VMEM((K,), jnp.int32) for _ in range(2)],
            [[pltpu.VMEM((K,), jnp.float32) for _ in range(NH)]
             for _ in range(2)],
            pltpu.VMEM((K,), jnp.int32),
            pltpu.VMEM((K,), jnp.float32),
            pltpu.VMEM((K,), jnp.float32),
            pltpu.VMEM((K,), jnp.float32),
            [pltpu.VMEM((K, NH * HD), jnp.float32) for _ in range(2)],
            [pltpu.VMEM((K, HD), jnp.float32) for _ in range(2)],
            pltpu.VMEM((K, HD), jnp.float32),
            pltpu.VMEM_SHARED((NP, HD), jnp.float32),
            [pltpu.SemaphoreType.DMA for _ in range(2)],
            [pltpu.SemaphoreType.DMA for _ in range(2)],
            pltpu.SemaphoreType.DMA,
        ],
        compiler_params=pltpu.CompilerParams(
            needs_layout_passes=False, use_tc_tiling_on_sc=False),
    )(_pass2_body)
    return f(src, dst, ewi, e0, e1, e2, Hhalf, et_half, rden)


def _final_body(lo_ref, hi_ref, out_ref):
    lo = lo_ref[0:N, :] + lo_ref[NP:NP + N, :]
    hi = hi_ref[0:N, :] + hi_ref[NP:NP + N, :]
    out_ref[...] = jnp.concatenate([lo, hi], axis=1)


def _final(outp_lo, outp_hi):
    return pl.pallas_call(
        _final_body,
        out_shape=jax.ShapeDtypeStruct((N, D), jnp.float32),
    )(outp_lo, outp_hi)


def kernel(x, edge_index, edge_weight, W_lin, edge_table, W_heads, a_src, a_dst):
    ei = edge_index.astype(jnp.int32)
    src = ei[0]
    dst = ei[1]
    ewi = edge_weight.astype(jnp.int32)
    Hlo, Hhi, alpha = _dense(x, W_lin, W_heads, a_src, a_dst)
    etlo, ethi = _etsplit(edge_table)
    e0, e1, e2, denp = _pass1(src, dst, alpha.reshape(N * 6))
    rden = _rden(denp)
    outp_lo = _pass2(src, dst, ewi, e0, e1, e2, Hlo, etlo, rden)
    outp_hi = _pass2(src, dst, ewi, e0, e1, e2, Hhi, ethi, rden)
    return _final(outp_lo, outp_hi)
